# baseline (speedup 1.0000x reference)
import functools

import jax
import jax.numpy as jnp
from jax import lax
from jax.experimental import pallas as pl
from jax.experimental.pallas import tpu as pltpu
from jax.experimental.pallas import tpu_sc as plsc

_NC = 2
_NS = 16
_FH = 64
_K = 80
_RG = 5
_RS = 2
_RL = 10

_F32 = jnp.float32
_BF16 = jnp.bfloat16


def _mm0_body(x_ref, w_ref, o_ref):
    x = x_ref[...]
    o_ref[0] = jnp.dot(x, w_ref[0], preferred_element_type=_F32).astype(_BF16)
    o_ref[1] = jnp.dot(x, w_ref[1], preferred_element_type=_F32).astype(_BF16)


def _tc_mm0(x, w2):
    return pl.pallas_call(
        _mm0_body,
        out_shape=jax.ShapeDtypeStruct((_NC, x.shape[0], _FH), _BF16),
    )(x, w2)


def _fused_body(n, p_ref, b_ref, w_ref, o_ref):
    h0 = jnp.maximum(p_ref[0, :n] + b_ref[0], 0.0)
    h1 = jnp.maximum(p_ref[1, :n] + b_ref[1], 0.0)
    for jo in range(2):
        o_ref[jo] = (
            jnp.dot(h0, w_ref[jo, 0], preferred_element_type=_F32)
            + jnp.dot(h1, w_ref[jo, 1], preferred_element_type=_F32)
        ).astype(_BF16)


def _tc_fused(p, b2, w4, n):
    return pl.pallas_call(
        functools.partial(_fused_body, n),
        out_shape=jax.ShapeDtypeStruct((_NC, n, _FH), _BF16),
    )(p, b2, w4)


def _relu_body(n, p_ref, b_ref, o_ref):
    o_ref[0] = jnp.maximum(p_ref[0, :n] + b_ref[0], 0.0).astype(_BF16)
    o_ref[1] = jnp.maximum(p_ref[1, :n] + b_ref[1], 0.0).astype(_BF16)


def _tc_relu(p, b2, n):
    return pl.pallas_call(
        functools.partial(_relu_body, n),
        out_shape=jax.ShapeDtypeStruct((_NC, n, _FH), _BF16),
    )(p, b2)


def _final_body(n, p_ref, w_ref, b_ref, o_ref):
    o_ref[...] = (
        jnp.dot(p_ref[0, :n], w_ref[0], preferred_element_type=_F32)
        + jnp.dot(p_ref[1, :n], w_ref[1], preferred_element_type=_F32)
        + b_ref[...]
    )


def _tc_final(p, w2, b, n):
    return pl.pallas_call(
        functools.partial(_final_body, n),
        out_shape=jax.ShapeDtypeStruct((n, w2.shape[2]), _F32),
    )(p, w2, b.reshape(1, -1))


@functools.cache
def _make_agg(npad, nchunks):
    rps = npad // _NS
    mesh = plsc.VectorSubcoreMesh(core_axis_name="c", subcore_axis_name="s")

    assert nchunks % _RL == 0 and nchunks >= 2 * _RL
    _LEAD = 7

    @functools.partial(
        pl.kernel,
        out_type=jax.ShapeDtypeStruct((_NC, npad, _FH), _F32),
        mesh=mesh,
        scratch_types=[
            [pltpu.VMEM((_K,), jnp.int32)] * _RL,
            [pltpu.VMEM((_K,), jnp.int32)] * _RL,
            [pltpu.VMEM((_K,), _F32)] * _RL,
            [pltpu.VMEM((_K, _FH), _BF16)] * _RG,
            [pltpu.VMEM((_K, _FH), _F32)] * _RS,
            pltpu.VMEM_SHARED((npad, _FH), _F32),
            [pltpu.SemaphoreType.DMA] * _RL,
            [pltpu.SemaphoreType.DMA] * _RG,
            [pltpu.SemaphoreType.DMA] * _RS,
        ],
        compiler_params=pltpu.CompilerParams(
            use_tc_tiling_on_sc=False, needs_layout_passes=False),
    )
    def agg(sup, srcs, dsts, ws, zeros, out, srcr, dstr, wr,
            gbufs, sbufs, acc, isems, gsems, ssems):
        c = lax.axis_index("c")
        s = lax.axis_index("s")
        mysrc, mydst, myw = srcs.at[s], dsts.at[s], ws.at[s]

        def fire_idx(j, slot):
            pltpu.async_copy(mysrc.at[j], srcr[slot], isems[slot])
            pltpu.async_copy(mydst.at[j], dstr[slot], isems[slot])
            pltpu.async_copy(myw.at[j], wr[slot], isems[slot])

        def wait_idx(j, slot):
            pltpu.make_async_copy(mysrc.at[j], srcr[slot], isems[slot]).wait()
            pltpu.make_async_copy(mydst.at[j], dstr[slot], isems[slot]).wait()
            pltpu.make_async_copy(myw.at[j], wr[slot], isems[slot]).wait()

        pltpu.sync_copy(zeros.at[pl.ds(s * rps, rps)], acc.at[pl.ds(s * rps, rps)])
        for j in range(_LEAD):
            fire_idx(j, j % _RL)
        for b in range(_RG):
            wait_idx(b, b % _RL)
        plsc.subcore_barrier()
        for b in range(_RG):
            pltpu.async_copy(sup.at[c].at[srcr[b % _RL]], gbufs[b], gsems[b])

        def block(iblk, carry):
            for b in range(_RL):
                i = iblk * _RL + b
                g, sg = gbufs[b % _RG], gsems[b % _RG]
                sb, ss = sbufs[b % _RS], ssems[b % _RS]
                pltpu.make_async_copy(sup.at[c].at[srcr[b]], g, sg).wait()
                @pl.when(i >= _RS)
                def _():
                    pltpu.make_async_copy(sb, acc.at[dstr[b]], ss).wait()

                def scale(gg, carry2):
                    wv = wr[b][pl.ds(gg * 16, 16)]
                    for lane in range(16):
                        wj = wv[lane]
                        row = gg * 16 + lane
                        for k2 in range(_FH // 32):
                            v = g[row, pl.ds(k2 * 32, 32)]
                            u = plsc.bitcast(v, jnp.uint32)
                            lo = plsc.bitcast(u << 16, _F32)
                            hi = plsc.bitcast(u & jnp.uint32(0xFFFF0000), _F32)
                            sb[row, pl.ds(k2 * 32, 16)] = lo * wj
                            sb[row, pl.ds(k2 * 32 + 16, 16)] = hi * wj
                    return carry2

                lax.fori_loop(0, _K // 16, scale, 0)
                pltpu.async_copy(sb, acc.at[dstr[b]], ss, add=True)
                @pl.when(i + _RG < nchunks)
                def _():
                    slot_g = (b + _RG) % _RL
                    wait_idx(i + _RG, slot_g)
                    pltpu.async_copy(sup.at[c].at[srcr[slot_g]], g, sg)
                @pl.when(i + _LEAD < nchunks)
                def _():
                    fire_idx(i + _LEAD, (b + _LEAD) % _RL)
            return carry

        lax.fori_loop(0, nchunks // _RL, block, 0)
        for b in range(_RS):
            pltpu.make_async_copy(sbufs[b], acc.at[dstr[0]], ssems[b]).wait()
        plsc.subcore_barrier()
        pltpu.sync_copy(acc.at[pl.ds(s * rps, rps)], out.at[c, pl.ds(s * rps, rps)])

    return agg


def kernel(features, edge_weight, W0, b0, W1, b1, W2, b2, edge_index):
    n = features.shape[0]
    e = edge_index.shape[1]

    src = edge_index[0]
    dst = edge_index[1]
    w = edge_weight

    per = _K * _NS
    nch = -(-e // per)
    nch = -(-nch // _RL) * _RL
    e_pad = nch * per
    if e_pad != e:
        pad = e_pad - e
        fill = (jnp.arange(pad, dtype=jnp.int32) % n).astype(jnp.int32)
        src = jnp.concatenate([src, fill])
        dst = jnp.concatenate([dst, fill])
        w = jnp.concatenate([w, jnp.zeros((pad,), _F32)])
    nchunks = e_pad // per

    srcs = src.reshape(_NS, nchunks, _K)
    dsts = dst.reshape(_NS, nchunks, _K)
    ws = w.reshape(_NS, nchunks, _K)

    npad = -(-n // (_NS * 8)) * (_NS * 8)
    zeros = jnp.zeros((npad, _FH), _F32)
    agg = _make_agg(npad, nchunks)

    d = W0.shape[0]
    q = []
    for k2 in range(_FH // 32):
        q += [32 * k2 + 2 * i for i in range(16)]
        q += [32 * k2 + 2 * i + 1 for i in range(16)]
    qq = [q[j] for j in q]
    qa = jnp.array(q, jnp.int32)
    qqa = jnp.array(qq, jnp.int32)
    rp1 = jnp.concatenate([qa, qa + _FH])
    rp2 = jnp.concatenate([qqa, qqa + _FH])

    w0s = W0.reshape(d, _NC, _FH).transpose(1, 0, 2)
    w1s = W1[rp1].reshape(_NC, _FH, _NC, _FH).transpose(2, 0, 1, 3)
    w2s = W2[rp2].reshape(_NC, _FH, W2.shape[1])
    b0s = b0.reshape(_NC, _FH)[:, qa].reshape(_NC, 1, _FH)
    b1s = b1.reshape(_NC, _FH)[:, qa].reshape(_NC, 1, _FH)

    s0 = _tc_mm0(features, w0s)
    p = agg(s0, srcs, dsts, ws, zeros)
    s1 = _tc_fused(p, b0s, w1s, n)
    p = agg(s1, srcs, dsts, ws, zeros)
    x2 = _tc_relu(p, b1s, n)
    p = agg(x2, srcs, dsts, ws, zeros)
    return _tc_final(p, w2s, b2, n)


# speedup vs baseline: 1.7699x; 1.7699x over previous
import functools

import jax
import jax.numpy as jnp
from jax import lax
from jax.experimental import pallas as pl
from jax.experimental.pallas import tpu as pltpu
from jax.experimental.pallas import tpu_sc as plsc

_NC = 2
_NS = 16
_FH = 64
_K = 80
_RG = 5
_RS = 2
_RL = 10

_F32 = jnp.float32


def _mm0_body(x_ref, w_ref, o_ref):
    x = x_ref[...]
    o_ref[0] = jnp.dot(x, w_ref[0], preferred_element_type=_F32)
    o_ref[1] = jnp.dot(x, w_ref[1], preferred_element_type=_F32)


def _tc_mm0(x, w2):
    return pl.pallas_call(
        _mm0_body,
        out_shape=jax.ShapeDtypeStruct((_NC, x.shape[0], _FH), _F32),
    )(x, w2)


def _fused_body(n, p_ref, b_ref, w_ref, o_ref):
    h0 = jnp.maximum(p_ref[0, :n] + b_ref[0], 0.0)
    h1 = jnp.maximum(p_ref[1, :n] + b_ref[1], 0.0)
    for jo in range(2):
        o_ref[jo] = jnp.dot(h0, w_ref[jo, 0], preferred_element_type=_F32) + jnp.dot(
            h1, w_ref[jo, 1], preferred_element_type=_F32
        )


def _tc_fused(p, b2, w4, n):
    return pl.pallas_call(
        functools.partial(_fused_body, n),
        out_shape=jax.ShapeDtypeStruct((_NC, n, _FH), _F32),
    )(p, b2, w4)


def _relu_body(n, p_ref, b_ref, o_ref):
    o_ref[0] = jnp.maximum(p_ref[0, :n] + b_ref[0], 0.0)
    o_ref[1] = jnp.maximum(p_ref[1, :n] + b_ref[1], 0.0)


def _tc_relu(p, b2, n):
    return pl.pallas_call(
        functools.partial(_relu_body, n),
        out_shape=jax.ShapeDtypeStruct((_NC, n, _FH), _F32),
    )(p, b2)


def _final_body(n, p_ref, w_ref, b_ref, o_ref):
    o_ref[...] = (
        jnp.dot(p_ref[0, :n], w_ref[0], preferred_element_type=_F32)
        + jnp.dot(p_ref[1, :n], w_ref[1], preferred_element_type=_F32)
        + b_ref[...]
    )


def _tc_final(p, w2, b, n):
    return pl.pallas_call(
        functools.partial(_final_body, n),
        out_shape=jax.ShapeDtypeStruct((n, w2.shape[2]), _F32),
    )(p, w2, b.reshape(1, -1))


@functools.cache
def _make_agg(npad, nchunks):
    rps = npad // _NS
    mesh = plsc.VectorSubcoreMesh(core_axis_name="c", subcore_axis_name="s")

    assert nchunks % _RL == 0 and nchunks >= 2 * _RL
    _LEAD = 7

    @functools.partial(
        pl.kernel,
        out_type=jax.ShapeDtypeStruct((_NC, npad, _FH), _F32),
        mesh=mesh,
        scratch_types=[
            [pltpu.VMEM((3, _K), jnp.int32)] * _RL,
            [pltpu.VMEM((_K, _FH), _F32)] * _RG,
            [pltpu.VMEM((_K, _FH), _F32)] * _RS,
            pltpu.VMEM_SHARED((npad, _FH), _F32),
            [pltpu.SemaphoreType.DMA] * _RL,
            [pltpu.SemaphoreType.DMA] * _RG,
            [pltpu.SemaphoreType.DMA] * _RS,
        ],
        compiler_params=pltpu.CompilerParams(
            use_tc_tiling_on_sc=False, needs_layout_passes=False),
    )
    def agg(sup, packed, zeros, out, idxr,
            gbufs, sbufs, acc, isems, gsems, ssems):
        c = lax.axis_index("c")
        s = lax.axis_index("s")
        mypk = packed.at[s]

        def fire_idx(j, slot):
            pltpu.async_copy(mypk.at[j], idxr[slot], isems[slot])

        def wait_idx(j, slot):
            pltpu.make_async_copy(mypk.at[j], idxr[slot], isems[slot]).wait()

        pltpu.sync_copy(zeros.at[pl.ds(s * rps, rps)], acc.at[pl.ds(s * rps, rps)])
        for j in range(_LEAD):
            fire_idx(j, j % _RL)
        for b in range(_RG):
            wait_idx(b, b % _RL)
        plsc.subcore_barrier()
        for b in range(_RG):
            pltpu.async_copy(sup.at[c].at[idxr[b % _RL].at[0]], gbufs[b], gsems[b])

        def block(iblk, carry):
            for b in range(_RL):
                i = iblk * _RL + b
                g, sg = gbufs[b % _RG], gsems[b % _RG]
                sb, ss = sbufs[b % _RS], ssems[b % _RS]
                pltpu.make_async_copy(sup.at[c].at[idxr[b].at[0]], g, sg).wait()
                @pl.when(i >= _RS)
                def _():
                    pltpu.make_async_copy(sb, acc.at[idxr[b].at[1]], ss).wait()

                def scale(gg, carry2):
                    wv = plsc.bitcast(idxr[b][2, pl.ds(gg * 16, 16)], _F32)
                    for lane in range(16):
                        wj = wv[lane]
                        row = gg * 16 + lane
                        for cb in range(_FH // 16):
                            sl = pl.ds(cb * 16, 16)
                            sb[row, sl] = g[row, sl] * wj
                    return carry2

                lax.fori_loop(0, _K // 16, scale, 0)
                pltpu.async_copy(sb, acc.at[idxr[b].at[1]], ss, add=True)
                @pl.when(i + _RG < nchunks)
                def _():
                    slot_g = (b + _RG) % _RL
                    wait_idx(i + _RG, slot_g)
                    pltpu.async_copy(sup.at[c].at[idxr[slot_g].at[0]], g, sg)
                @pl.when(i + _LEAD < nchunks)
                def _():
                    fire_idx(i + _LEAD, (b + _LEAD) % _RL)
            return carry

        lax.fori_loop(0, nchunks // _RL, block, 0)
        for b in range(_RS):
            pltpu.make_async_copy(sbufs[b], acc.at[idxr[0].at[1]], ssems[b]).wait()
        plsc.subcore_barrier()
        pltpu.sync_copy(acc.at[pl.ds(s * rps, rps)], out.at[c, pl.ds(s * rps, rps)])

    return agg


def kernel(features, edge_weight, W0, b0, W1, b1, W2, b2, edge_index):
    n = features.shape[0]
    e = edge_index.shape[1]

    src = edge_index[0]
    dst = edge_index[1]
    w = edge_weight

    per = _K * _NS
    nch = -(-e // per)
    nch = -(-nch // _RL) * _RL
    e_pad = nch * per
    if e_pad != e:
        pad = e_pad - e
        fill = (jnp.arange(pad, dtype=jnp.int32) % n).astype(jnp.int32)
        src = jnp.concatenate([src, fill])
        dst = jnp.concatenate([dst, fill])
        w = jnp.concatenate([w, jnp.zeros((pad,), _F32)])
    nchunks = e_pad // per

    wbits = lax.bitcast_convert_type(w, jnp.int32)
    packed = jnp.stack(
        [src.reshape(_NS, nchunks, _K), dst.reshape(_NS, nchunks, _K),
         wbits.reshape(_NS, nchunks, _K)], axis=2)

    npad = -(-n // (_NS * 8)) * (_NS * 8)
    zeros = jnp.zeros((npad, _FH), _F32)
    agg = _make_agg(npad, nchunks)

    d = W0.shape[0]
    w0s = W0.reshape(d, _NC, _FH).transpose(1, 0, 2)
    w1s = W1.reshape(_NC, _FH, _NC, _FH).transpose(2, 0, 1, 3)
    w2s = W2.reshape(_NC, _FH, W2.shape[1])
    b0s = b0.reshape(_NC, 1, _FH)
    b1s = b1.reshape(_NC, 1, _FH)

    s0 = _tc_mm0(features, w0s)
    p = agg(s0, packed, zeros)
    s1 = _tc_fused(p, b0s, w1s, n)
    p = agg(s1, packed, zeros)
    x2 = _tc_relu(p, b1s, n)
    p = agg(x2, packed, zeros)
    return _tc_final(p, w2s, b2, n)


# speedup vs baseline: 1.8077x; 1.0214x over previous
import functools

import jax
import jax.numpy as jnp
from jax import lax
from jax.experimental import pallas as pl
from jax.experimental.pallas import tpu as pltpu
from jax.experimental.pallas import tpu_sc as plsc

_NC = 2
_NS = 16
_FH = 64
_K = 80
_RG = 5
_RS = 2
_RL = 10

_F32 = jnp.float32


def _mm0_body(x_ref, w_ref, o_ref):
    x = x_ref[...]
    o_ref[0] = jnp.dot(x, w_ref[0], preferred_element_type=_F32)
    o_ref[1] = jnp.dot(x, w_ref[1], preferred_element_type=_F32)


def _tc_mm0(x, w2):
    return pl.pallas_call(
        _mm0_body,
        out_shape=jax.ShapeDtypeStruct((_NC, x.shape[0], _FH), _F32),
    )(x, w2)


def _fused_body(n, p_ref, b_ref, w_ref, o_ref):
    h0 = jnp.maximum(p_ref[0, :n] + b_ref[0], 0.0)
    h1 = jnp.maximum(p_ref[1, :n] + b_ref[1], 0.0)
    for jo in range(2):
        o_ref[jo] = jnp.dot(h0, w_ref[jo, 0], preferred_element_type=_F32) + jnp.dot(
            h1, w_ref[jo, 1], preferred_element_type=_F32
        )


def _tc_fused(p, b2, w4, n):
    return pl.pallas_call(
        functools.partial(_fused_body, n),
        out_shape=jax.ShapeDtypeStruct((_NC, n, _FH), _F32),
    )(p, b2, w4)


def _relu_body(n, p_ref, b_ref, o_ref):
    o_ref[0] = jnp.maximum(p_ref[0, :n] + b_ref[0], 0.0)
    o_ref[1] = jnp.maximum(p_ref[1, :n] + b_ref[1], 0.0)


def _tc_relu(p, b2, n):
    return pl.pallas_call(
        functools.partial(_relu_body, n),
        out_shape=jax.ShapeDtypeStruct((_NC, n, _FH), _F32),
    )(p, b2)


def _final_body(n, p_ref, w_ref, b_ref, o_ref):
    o_ref[...] = (
        jnp.dot(p_ref[0, :n], w_ref[0], preferred_element_type=_F32)
        + jnp.dot(p_ref[1, :n], w_ref[1], preferred_element_type=_F32)
        + b_ref[...]
    )


def _tc_final(p, w2, b, n):
    return pl.pallas_call(
        functools.partial(_final_body, n),
        out_shape=jax.ShapeDtypeStruct((n, w2.shape[2]), _F32),
    )(p, w2, b.reshape(1, -1))


@functools.cache
def _make_agg(npad, nchunks):
    rps = npad // _NS
    mesh = plsc.VectorSubcoreMesh(core_axis_name="c", subcore_axis_name="s")

    assert nchunks % _RL == 0 and nchunks >= 2 * _RL
    _LEAD = 7

    @functools.partial(
        pl.kernel,
        out_type=jax.ShapeDtypeStruct((_NC, npad, _FH), _F32),
        mesh=mesh,
        scratch_types=[
            [pltpu.VMEM((2, _K), jnp.int32)] * _RL,
            [pltpu.VMEM((_K,), _F32)] * _RL,
            [pltpu.VMEM((_K, _FH), _F32)] * _RG,
            [pltpu.VMEM((_K, _FH), _F32)] * _RS,
            pltpu.VMEM_SHARED((npad, _FH), _F32),
            [pltpu.SemaphoreType.DMA] * _RL,
            [pltpu.SemaphoreType.DMA] * _RG,
            [pltpu.SemaphoreType.DMA] * _RS,
        ],
        compiler_params=pltpu.CompilerParams(use_tc_tiling_on_sc=False),
    )
    def agg(sup, sds, ws, zeros, out, idxr, wr,
            gbufs, sbufs, acc, isems, gsems, ssems):
        c = lax.axis_index("c")
        s = lax.axis_index("s")
        mysd, myw = sds.at[s], ws.at[s]

        def fire_idx(j, slot):
            pltpu.async_copy(mysd.at[j], idxr[slot], isems[slot])
            pltpu.async_copy(myw.at[j], wr[slot], isems[slot])

        def wait_idx(j, slot):
            pltpu.make_async_copy(mysd.at[j], idxr[slot], isems[slot]).wait()
            pltpu.make_async_copy(myw.at[j], wr[slot], isems[slot]).wait()

        pltpu.sync_copy(zeros.at[pl.ds(s * rps, rps)], acc.at[pl.ds(s * rps, rps)])
        for j in range(_LEAD):
            fire_idx(j, j % _RL)
        for b in range(_RG):
            wait_idx(b, b % _RL)
        plsc.subcore_barrier()
        for b in range(_RG):
            pltpu.async_copy(sup.at[c].at[idxr[b % _RL].at[0]], gbufs[b], gsems[b])

        def block(iblk, carry):
            for b in range(_RL):
                i = iblk * _RL + b
                g, sg = gbufs[b % _RG], gsems[b % _RG]
                sb, ss = sbufs[b % _RS], ssems[b % _RS]
                pltpu.make_async_copy(sup.at[c].at[idxr[b].at[0]], g, sg).wait()
                @pl.when(i >= _RS)
                def _():
                    pltpu.make_async_copy(sb, acc.at[idxr[b].at[1]], ss).wait()

                def scale(gg, carry2):
                    wv = wr[b][pl.ds(gg * 16, 16)]
                    for lane in range(16):
                        wj = wv[lane]
                        row = gg * 16 + lane
                        for cb in range(_FH // 16):
                            sl = pl.ds(cb * 16, 16)
                            sb[row, sl] = g[row, sl] * wj
                    return carry2

                lax.fori_loop(0, _K // 16, scale, 0)
                pltpu.async_copy(sb, acc.at[idxr[b].at[1]], ss, add=True)
                @pl.when(i + _RG < nchunks)
                def _():
                    slot_g = (b + _RG) % _RL
                    wait_idx(i + _RG, slot_g)
                    pltpu.async_copy(sup.at[c].at[idxr[slot_g].at[0]], g, sg)
                @pl.when(i + _LEAD < nchunks)
                def _():
                    fire_idx(i + _LEAD, (b + _LEAD) % _RL)
            return carry

        lax.fori_loop(0, nchunks // _RL, block, 0)
        for b in range(_RS):
            pltpu.make_async_copy(sbufs[b], acc.at[idxr[0].at[1]], ssems[b]).wait()
        plsc.subcore_barrier()
        pltpu.sync_copy(acc.at[pl.ds(s * rps, rps)], out.at[c, pl.ds(s * rps, rps)])

    return agg


def kernel(features, edge_weight, W0, b0, W1, b1, W2, b2, edge_index):
    n = features.shape[0]
    e = edge_index.shape[1]

    src = edge_index[0]
    dst = edge_index[1]
    w = edge_weight

    per = _K * _NS
    nch = -(-e // per)
    nch = -(-nch // _RL) * _RL
    e_pad = nch * per
    if e_pad != e:
        pad = e_pad - e
        fill = (jnp.arange(pad, dtype=jnp.int32) % n).astype(jnp.int32)
        src = jnp.concatenate([src, fill])
        dst = jnp.concatenate([dst, fill])
        w = jnp.concatenate([w, jnp.zeros((pad,), _F32)])
    nchunks = e_pad // per

    sds = jnp.stack(
        [src.reshape(_NS, nchunks, _K), dst.reshape(_NS, nchunks, _K)], axis=2)
    ws = w.reshape(_NS, nchunks, _K)

    npad = -(-n // (_NS * 8)) * (_NS * 8)
    zeros = jnp.zeros((npad, _FH), _F32)
    agg = _make_agg(npad, nchunks)

    d = W0.shape[0]
    w0s = W0.reshape(d, _NC, _FH).transpose(1, 0, 2)
    w1s = W1.reshape(_NC, _FH, _NC, _FH).transpose(2, 0, 1, 3)
    w2s = W2.reshape(_NC, _FH, W2.shape[1])
    b0s = b0.reshape(_NC, 1, _FH)
    b1s = b1.reshape(_NC, 1, _FH)

    s0 = _tc_mm0(features, w0s)
    p = agg(s0, sds, ws, zeros)
    s1 = _tc_fused(p, b0s, w1s, n)
    p = agg(s1, sds, ws, zeros)
    x2 = _tc_relu(p, b1s, n)
    p = agg(x2, sds, ws, zeros)
    return _tc_final(p, w2s, b2, n)


# speedup vs baseline: 1.9085x; 1.0557x over previous
import functools

import jax
import jax.numpy as jnp
from jax import lax
from jax.experimental import pallas as pl
from jax.experimental.pallas import tpu as pltpu
from jax.experimental.pallas import tpu_sc as plsc

_NC = 2
_NS = 16
_FH = 64
_K = 80
_RG = 5
_RS = 2
_RL = 10

_F32 = jnp.float32


def _mm0_body(x_ref, w_ref, o_ref):
    x = x_ref[...]
    o_ref[0] = jnp.dot(x, w_ref[0], preferred_element_type=_F32)
    o_ref[1] = jnp.dot(x, w_ref[1], preferred_element_type=_F32)


def _tc_mm0(x, w2):
    return pl.pallas_call(
        _mm0_body,
        out_shape=jax.ShapeDtypeStruct((_NC, x.shape[0], _FH), _F32),
    )(x, w2)


def _fused_body(n, p_ref, b_ref, w_ref, o_ref):
    h0 = jnp.maximum(p_ref[0, :n] + b_ref[0], 0.0)
    h1 = jnp.maximum(p_ref[1, :n] + b_ref[1], 0.0)
    for jo in range(2):
        o_ref[jo] = jnp.dot(h0, w_ref[jo, 0], preferred_element_type=_F32) + jnp.dot(
            h1, w_ref[jo, 1], preferred_element_type=_F32
        )


def _tc_fused(p, b2, w4, n):
    return pl.pallas_call(
        functools.partial(_fused_body, n),
        out_shape=jax.ShapeDtypeStruct((_NC, n, _FH), _F32),
    )(p, b2, w4)


def _relu_body(n, p_ref, b_ref, o_ref):
    o_ref[0] = jnp.maximum(p_ref[0, :n] + b_ref[0], 0.0)
    o_ref[1] = jnp.maximum(p_ref[1, :n] + b_ref[1], 0.0)


def _tc_relu(p, b2, n):
    return pl.pallas_call(
        functools.partial(_relu_body, n),
        out_shape=jax.ShapeDtypeStruct((_NC, n, _FH), _F32),
    )(p, b2)


def _final_body(n, p_ref, w_ref, b_ref, o_ref):
    o_ref[...] = (
        jnp.dot(p_ref[0, :n], w_ref[0], preferred_element_type=_F32)
        + jnp.dot(p_ref[1, :n], w_ref[1], preferred_element_type=_F32)
        + b_ref[...]
    )


def _tc_final(p, w2, b, n):
    return pl.pallas_call(
        functools.partial(_final_body, n),
        out_shape=jax.ShapeDtypeStruct((n, w2.shape[2]), _F32),
    )(p, w2, b.reshape(1, -1))


@functools.cache
def _make_agg(npad, nchunks):
    rps = npad // _NS
    mesh = plsc.VectorSubcoreMesh(core_axis_name="c", subcore_axis_name="s")

    assert nchunks % _RL == 0 and nchunks >= 2 * _RL
    _LEAD = 7

    @functools.partial(
        pl.kernel,
        out_type=jax.ShapeDtypeStruct((_NC, npad, _FH), _F32),
        mesh=mesh,
        scratch_types=[
            [pltpu.VMEM((_K,), jnp.int32)] * _RL,
            [pltpu.VMEM((_K,), jnp.int32)] * _RL,
            [pltpu.VMEM((_K,), _F32)] * _RL,
            [pltpu.VMEM((_K, _FH), _F32)] * _RG,
            [pltpu.VMEM((_K, _FH), _F32)] * _RS,
            pltpu.VMEM_SHARED((npad, _FH), _F32),
            [pltpu.SemaphoreType.DMA] * _RL,
            [pltpu.SemaphoreType.DMA] * _RG,
            [pltpu.SemaphoreType.DMA] * _RS,
        ],
        compiler_params=pltpu.CompilerParams(use_tc_tiling_on_sc=False),
    )
    def agg(sup, srcs, dsts, ws, zeros, out, srcr, dstr, wr,
            gbufs, sbufs, acc, isems, gsems, ssems):
        c = lax.axis_index("c")
        s = lax.axis_index("s")
        mysrc, mydst, myw = srcs.at[s], dsts.at[s], ws.at[s]

        def fire_idx(j, slot):
            pltpu.async_copy(mysrc.at[j], srcr[slot], isems[slot])
            pltpu.async_copy(mydst.at[j], dstr[slot], isems[slot])
            pltpu.async_copy(myw.at[j], wr[slot], isems[slot])

        def wait_idx(j, slot):
            pltpu.make_async_copy(mysrc.at[j], srcr[slot], isems[slot]).wait()
            pltpu.make_async_copy(mydst.at[j], dstr[slot], isems[slot]).wait()
            pltpu.make_async_copy(myw.at[j], wr[slot], isems[slot]).wait()

        pltpu.sync_copy(zeros.at[pl.ds(s * rps, rps)], acc.at[pl.ds(s * rps, rps)])
        for j in range(_LEAD):
            fire_idx(j, j % _RL)
        for b in range(_RG):
            wait_idx(b, b % _RL)
        plsc.subcore_barrier()
        for b in range(_RG):
            pltpu.async_copy(sup.at[c].at[srcr[b % _RL]], gbufs[b], gsems[b])

        def block(iblk, carry):
            for b in range(_RL):
                i = iblk * _RL + b
                g, sg = gbufs[b % _RG], gsems[b % _RG]
                sb, ss = sbufs[b % _RS], ssems[b % _RS]
                pltpu.make_async_copy(sup.at[c].at[srcr[b]], g, sg).wait()
                @pl.when(i >= _RS)
                def _():
                    pltpu.make_async_copy(sb, acc.at[dstr[b]], ss).wait()

                def scale(gg, carry2):
                    wv = wr[b][pl.ds(gg * 16, 16)]
                    for lane in range(16):
                        wj = wv[lane]
                        row = gg * 16 + lane
                        for cb in range(_FH // 16):
                            sl = pl.ds(cb * 16, 16)
                            sb[row, sl] = g[row, sl] * wj
                    return carry2

                lax.fori_loop(0, _K // 16, scale, 0)
                pltpu.async_copy(sb, acc.at[dstr[b]], ss, add=True)
                @pl.when(i + _RG < nchunks)
                def _():
                    slot_g = (b + _RG) % _RL
                    wait_idx(i + _RG, slot_g)
                    pltpu.async_copy(sup.at[c].at[srcr[slot_g]], g, sg)
                @pl.when(i + _LEAD < nchunks)
                def _():
                    fire_idx(i + _LEAD, (b + _LEAD) % _RL)
            return carry

        lax.fori_loop(0, nchunks // _RL, block, 0)
        for b in range(_RS):
            pltpu.make_async_copy(sbufs[b], acc.at[dstr[0]], ssems[b]).wait()
        plsc.subcore_barrier()
        pltpu.sync_copy(acc.at[pl.ds(s * rps, rps)], out.at[c, pl.ds(s * rps, rps)])

    return agg


def kernel(features, edge_weight, W0, b0, W1, b1, W2, b2, edge_index):
    n = features.shape[0]
    e = edge_index.shape[1]

    src = edge_index[0]
    dst = edge_index[1]
    w = edge_weight

    per = _K * _NS
    nch = -(-e // per)
    nch = -(-nch // _RL) * _RL
    e_pad = nch * per
    if e_pad != e:
        pad = e_pad - e
        fill = (jnp.arange(pad, dtype=jnp.int32) % n).astype(jnp.int32)
        src = jnp.concatenate([src, fill])
        dst = jnp.concatenate([dst, fill])
        w = jnp.concatenate([w, jnp.zeros((pad,), _F32)])
    nchunks = e_pad // per

    srcs = src.reshape(_NS, nchunks, _K)
    dsts = dst.reshape(_NS, nchunks, _K)
    ws = w.reshape(_NS, nchunks, _K)

    npad = -(-n // (_NS * 8)) * (_NS * 8)
    zeros = jnp.zeros((npad, _FH), _F32)
    agg = _make_agg(npad, nchunks)

    d = W0.shape[0]
    w0s = W0.reshape(d, _NC, _FH).transpose(1, 0, 2)
    w1s = W1.reshape(_NC, _FH, _NC, _FH).transpose(2, 0, 1, 3)
    w2s = W2.reshape(_NC, _FH, W2.shape[1])
    b0s = b0.reshape(_NC, 1, _FH)
    b1s = b1.reshape(_NC, 1, _FH)

    s0 = _tc_mm0(features, w0s)
    p = agg(s0, srcs, dsts, ws, zeros)
    s1 = _tc_fused(p, b0s, w1s, n)
    p = agg(s1, srcs, dsts, ws, zeros)
    x2 = _tc_relu(p, b1s, n)
    p = agg(x2, srcs, dsts, ws, zeros)
    return _tc_final(p, w2s, b2, n)
